# baseline (speedup 1.0000x reference)
import jax
import jax.numpy as jnp
from jax.experimental import pallas as pl

_B, _N, _D = 16, 1024, 32
_BN = _B * _N
_K = 8192
_RB = 128
_NB = _BN // _RB


def _vq_kernel(x_ref, et_ref, q_ref, ind_ref, loss_ref):
    i = pl.program_id(0)

    xb = x_ref[...]
    et = et_ref[...]

    x_sq = jnp.sum(xb * xb, axis=1, keepdims=True)
    e_sq = jnp.sum(et * et, axis=0, keepdims=True)
    xe = jax.lax.dot_general(
        xb.astype(jnp.bfloat16), et.astype(jnp.bfloat16),
        (((1,), (0,)), ((), ())),
        preferred_element_type=jnp.float32)
    dist = -(x_sq - 2.0 * xe + e_sq)

    _C = 4096
    iota = jax.lax.broadcasted_iota(jnp.int32, (_RB, _K), 1)
    best = jnp.full((_RB, 1), -jnp.inf, jnp.float32)
    ind = jnp.zeros((_RB, 1), jnp.int32)
    for c in range(_K // _C):
        seg = dist[:, c * _C:(c + 1) * _C]
        seg_iota = iota[:, c * _C:(c + 1) * _C]
        m = jnp.max(seg, axis=1, keepdims=True)
        mi = jnp.min(jnp.where(seg == m, seg_iota, _K),
                     axis=1, keepdims=True)
        upd = m > best
        best = jnp.where(upd, m, best)
        ind = jnp.where(upd, mi, ind)
        best = best.astype(jnp.bfloat16).astype(jnp.float32)
    ind = ind[:, 0]

    onehot = (iota == ind[:, None]).astype(jnp.float32)
    qb = jax.lax.dot_general(
        onehot, et, (((1,), (1,)), ((), ())),
        precision=jax.lax.Precision.HIGHEST,
        preferred_element_type=jnp.float32)

    q_ref[...] = qb
    ind_ref[0, 0, :] = ind

    @pl.when(i == 0)
    def _init():
        loss_ref[...] = jnp.zeros((1, 1), jnp.float32)

    diff = qb - xb
    loss_ref[...] += jnp.sum(diff * diff).reshape(1, 1)


@jax.jit
def kernel(x, embed):
    B, N, D = x.shape
    flat = x.reshape(_BN, D)
    et = embed.T

    q, ind3, loss = pl.pallas_call(
        _vq_kernel,
        grid=(_NB,),
        in_specs=[
            pl.BlockSpec((_RB, _D), lambda i: (i, 0)),
            pl.BlockSpec((_D, _K), lambda i: (0, 0)),
        ],
        out_specs=[
            pl.BlockSpec((_RB, _D), lambda i: (i, 0)),
            pl.BlockSpec((1, 1, _RB), lambda i: (i, 0, 0)),
            pl.BlockSpec((1, 1), lambda i: (0, 0)),
        ],
        out_shape=[
            jax.ShapeDtypeStruct((_BN, _D), jnp.float32),
            jax.ShapeDtypeStruct((_NB, 1, _RB), jnp.int32),
            jax.ShapeDtypeStruct((1, 1), jnp.float32),
        ],
    )(flat, et)

    quantize = q.reshape(B, N, D)
    embed_ind = ind3.reshape(_BN).reshape(B, N)
    commit_loss = loss[0, 0] / (_BN * _D)
    return quantize, embed_ind, commit_loss


# speedup vs baseline: 1.0829x; 1.0829x over previous
import jax
import jax.numpy as jnp
from jax.experimental import pallas as pl

_B, _N, _D = 16, 1024, 32
_BN = _B * _N
_K = 8192
_RB = 512
_NB = _BN // _RB


def _vq_kernel(x_ref, et_ref, q_ref, ind_ref, loss_ref):
    i = pl.program_id(0)

    xb = x_ref[...]
    et = et_ref[...]

    x_sq = jnp.sum(xb * xb, axis=1, keepdims=True)
    e_sq = jnp.sum(et * et, axis=0, keepdims=True)
    xe = jax.lax.dot_general(
        xb.astype(jnp.bfloat16), et.astype(jnp.bfloat16),
        (((1,), (0,)), ((), ())),
        preferred_element_type=jnp.float32)
    dist = -(x_sq - 2.0 * xe + e_sq)

    _C = 4096
    iota = jax.lax.broadcasted_iota(jnp.int32, (_RB, _K), 1)
    best = jnp.full((_RB, 1), -jnp.inf, jnp.float32)
    ind = jnp.zeros((_RB, 1), jnp.int32)
    for c in range(_K // _C):
        seg = dist[:, c * _C:(c + 1) * _C]
        seg_iota = iota[:, c * _C:(c + 1) * _C]
        m = jnp.max(seg, axis=1, keepdims=True)
        mi = jnp.min(jnp.where(seg == m, seg_iota, _K),
                     axis=1, keepdims=True)
        upd = m > best
        best = jnp.where(upd, m, best)
        ind = jnp.where(upd, mi, ind)
        best = best.astype(jnp.bfloat16).astype(jnp.float32)
    ind = ind[:, 0]

    onehot = (iota == ind[:, None]).astype(jnp.float32)
    qb = jax.lax.dot_general(
        onehot, et, (((1,), (1,)), ((), ())),
        precision=jax.lax.Precision.HIGHEST,
        preferred_element_type=jnp.float32)

    q_ref[...] = qb
    ind_ref[0, 0, :] = ind

    @pl.when(i == 0)
    def _init():
        loss_ref[...] = jnp.zeros((1, 1), jnp.float32)

    diff = qb - xb
    loss_ref[...] += jnp.sum(diff * diff).reshape(1, 1)


@jax.jit
def kernel(x, embed):
    B, N, D = x.shape
    flat = x.reshape(_BN, D)
    et = embed.T

    q, ind3, loss = pl.pallas_call(
        _vq_kernel,
        grid=(_NB,),
        in_specs=[
            pl.BlockSpec((_RB, _D), lambda i: (i, 0)),
            pl.BlockSpec((_D, _K), lambda i: (0, 0)),
        ],
        out_specs=[
            pl.BlockSpec((_RB, _D), lambda i: (i, 0)),
            pl.BlockSpec((1, 1, _RB), lambda i: (i, 0, 0)),
            pl.BlockSpec((1, 1), lambda i: (0, 0)),
        ],
        out_shape=[
            jax.ShapeDtypeStruct((_BN, _D), jnp.float32),
            jax.ShapeDtypeStruct((_NB, 1, _RB), jnp.int32),
            jax.ShapeDtypeStruct((1, 1), jnp.float32),
        ],
    )(flat, et)

    quantize = q.reshape(B, N, D)
    embed_ind = ind3.reshape(_BN).reshape(B, N)
    commit_loss = loss[0, 0] / (_BN * _D)
    return quantize, embed_ind, commit_loss
